# baseline (speedup 1.0000x reference)
import functools

import jax
import jax.numpy as jnp
from jax import lax
from jax.experimental import pallas as pl
from jax.experimental.pallas import tpu as pltpu

N_EMB = 1024
DIM = 64
T_TOK = 16384
BLK = 512
N_BLK = T_TOK // BLK
TBLK = BLK // 16
DECAY_C = 0.99
COMMIT_C = 0.25
EPS_C = 1e-05


def _pass1_body(inp_ref, emb_ref, dist_ref, enc_ref, idx_ref, cnt_ref, dw_ref):
    i = pl.program_id(0)
    x2d = inp_ref[...].reshape(DIM, BLK)
    e = emb_ref[...]
    flat = x2d.T
    xs = jnp.sum(flat * flat, axis=1, keepdims=True)
    es = jnp.sum(e * e, axis=1)
    xe = lax.dot_general(x2d, e, (((0,), (1,)), ((), ())))
    dist = xs + es[None, :] - 2.0 * xe
    dist_ref[...] = dist
    idx = jnp.argmin(dist, axis=1).astype(jnp.int32)
    idx_ref[...] = idx.reshape(1, 1, BLK)
    cols = lax.broadcasted_iota(jnp.int32, (BLK, N_EMB), 1)
    ench = (cols == idx[:, None]).astype(jnp.bfloat16)
    enc_ref[...] = ench.astype(jnp.float32)
    ones8 = jnp.ones((8, BLK), jnp.bfloat16)
    cnt = lax.dot_general(ones8, ench, (((1,), (0,)), ((), ())),
                          preferred_element_type=jnp.float32)[0]
    fh = x2d.astype(jnp.bfloat16)
    fl = (x2d - fh.astype(jnp.float32)).astype(jnp.bfloat16)
    dwp = (lax.dot_general(ench, fh, (((0,), (1,)), ((), ())),
                           preferred_element_type=jnp.float32)
           + lax.dot_general(ench, fl, (((0,), (1,)), ((), ())),
                             preferred_element_type=jnp.float32))

    @pl.when(i == 0)
    def _():
        cnt_ref[...] = cnt[None, :]
        dw_ref[...] = dwp

    @pl.when(i > 0)
    def _():
        cnt_ref[...] += cnt[None, :]
        dw_ref[...] += dwp


def _pass2_body(inp_ref, idx_ref, cnt_ref, dw_ref, emaw_ref, emacs_ref,
                q_ref, loss_ref, perp_ref, w_scr, acc_scr):
    i = pl.program_id(0)

    @pl.when(i == 0)
    def _():
        counts = cnt_ref[0, :]
        cs = emacs_ref[0, :] * DECAY_C + (1.0 - DECAY_C) * counts
        n = jnp.sum(cs)
        csn = (cs + EPS_C) / (n + N_EMB * EPS_C) * n
        w = (emaw_ref[...] * DECAY_C + (1.0 - DECAY_C) * dw_ref[...])
        w_scr[...] = w / csn[:, None]
        p = counts * (1.0 / T_TOK)
        perp_ref[...] = jnp.exp(-jnp.sum(p * jnp.log(p + 1e-10))).reshape(1, 1)
        acc_scr[...] = jnp.zeros((1, 1), jnp.float32)

    idx = idx_ref[0, 0, :]
    cols = lax.broadcasted_iota(jnp.int32, (BLK, N_EMB), 1)
    ench = (cols == idx[:, None]).astype(jnp.bfloat16)
    w = w_scr[...]
    wh = w.astype(jnp.bfloat16)
    wl = (w - wh.astype(jnp.float32)).astype(jnp.bfloat16)
    q = (lax.dot_general(ench, wh, (((1,), (0,)), ((), ())),
                         preferred_element_type=jnp.float32)
         + lax.dot_general(ench, wl, (((1,), (0,)), ((), ())),
                           preferred_element_type=jnp.float32))
    x3 = inp_ref[...]
    flat = x3.reshape(DIM, BLK).T
    d = q - flat
    acc_scr[...] = acc_scr[...] + jnp.sum(d * d).reshape(1, 1)
    q_ref[...] = q.T.reshape(DIM, TBLK, 16)

    @pl.when(i == N_BLK - 1)
    def _():
        loss_ref[...] = acc_scr[...] * (COMMIT_C / (T_TOK * DIM))


@jax.jit
def kernel(inputs, embedding_weight, ema_w, ema_cluster_size):
    dist, enc, idx, cnt, dw = pl.pallas_call(
        _pass1_body,
        grid=(N_BLK,),
        in_specs=[
            pl.BlockSpec((DIM, TBLK, 16), lambda i: (0, i, 0)),
            pl.BlockSpec((N_EMB, DIM), lambda i: (0, 0)),
        ],
        out_specs=[
            pl.BlockSpec((BLK, N_EMB), lambda i: (i, 0)),
            pl.BlockSpec((BLK, N_EMB), lambda i: (i, 0)),
            pl.BlockSpec((1, 1, BLK), lambda i: (i, 0, 0)),
            pl.BlockSpec((1, N_EMB), lambda i: (0, 0)),
            pl.BlockSpec((N_EMB, DIM), lambda i: (0, 0)),
        ],
        out_shape=[
            jax.ShapeDtypeStruct((T_TOK, N_EMB), jnp.float32),
            jax.ShapeDtypeStruct((T_TOK, N_EMB), jnp.float32),
            jax.ShapeDtypeStruct((N_BLK, 1, BLK), jnp.int32),
            jax.ShapeDtypeStruct((1, N_EMB), jnp.float32),
            jax.ShapeDtypeStruct((N_EMB, DIM), jnp.float32),
        ],
    )(inputs, embedding_weight)

    q, loss, perp = pl.pallas_call(
        _pass2_body,
        grid=(N_BLK,),
        in_specs=[
            pl.BlockSpec((DIM, TBLK, 16), lambda i: (0, i, 0)),
            pl.BlockSpec((1, 1, BLK), lambda i: (i, 0, 0)),
            pl.BlockSpec((1, N_EMB), lambda i: (0, 0)),
            pl.BlockSpec((N_EMB, DIM), lambda i: (0, 0)),
            pl.BlockSpec((N_EMB, DIM), lambda i: (0, 0)),
            pl.BlockSpec((1, N_EMB), lambda i: (0, 0)),
        ],
        out_specs=[
            pl.BlockSpec((DIM, TBLK, 16), lambda i: (0, i, 0)),
            pl.BlockSpec((1, 1), lambda i: (0, 0)),
            pl.BlockSpec((1, 1), lambda i: (0, 0)),
        ],
        out_shape=[
            jax.ShapeDtypeStruct((DIM, 1024, 16), jnp.float32),
            jax.ShapeDtypeStruct((1, 1), jnp.float32),
            jax.ShapeDtypeStruct((1, 1), jnp.float32),
        ],
        scratch_shapes=[
            pltpu.VMEM((N_EMB, DIM), jnp.float32),
            pltpu.VMEM((1, 1), jnp.float32),
        ],
    )(inputs, idx, cnt, dw, ema_w, ema_cluster_size.reshape(1, N_EMB))

    return (loss[0, 0], q, perp[0, 0], enc, dist)


# speedup vs baseline: 1.7644x; 1.7644x over previous
import functools

import jax
import jax.numpy as jnp
from jax import lax
from jax.experimental import pallas as pl
from jax.experimental.pallas import tpu as pltpu

N_EMB = 1024
DIM = 64
T_TOK = 16384
BLK = 512
N_BLK = T_TOK // BLK
DECAY_C = 0.99
COMMIT_C = 0.25
EPS_C = 1e-05


def _pass1_body(inp_ref, emb_ref, dist_ref, enc_ref, idx_ref, cnt_ref,
                dwt_ref, es_scr):
    i = pl.program_id(0)
    x2d = inp_ref[...]
    e = emb_ref[...]

    @pl.when(i == 0)
    def _():
        es_scr[...] = jnp.sum(e * e, axis=1, keepdims=True).T

    flat = x2d.T
    xs = jnp.sum(flat * flat, axis=1, keepdims=True)
    xe = lax.dot_general(x2d, e, (((0,), (1,)), ((), ())))
    dist = xs + es_scr[...] - 2.0 * xe
    dist_ref[...] = dist
    idx = jnp.argmin(dist, axis=1).astype(jnp.int32)
    idx_ref[...] = idx.reshape(1, 1, BLK)
    cols = lax.broadcasted_iota(jnp.int32, (BLK, N_EMB), 1)
    ench = (cols == idx[:, None]).astype(jnp.bfloat16)
    enc_ref[...] = ench.astype(jnp.float32)
    ones8 = jnp.ones((8, BLK), jnp.bfloat16)
    cnt = lax.dot_general(ones8, ench, (((1,), (0,)), ((), ())),
                          preferred_element_type=jnp.float32)[0]
    fh = x2d.astype(jnp.bfloat16)
    fl = (x2d - fh.astype(jnp.float32)).astype(jnp.bfloat16)
    dwt = (lax.dot_general(fh, ench, (((1,), (0,)), ((), ())),
                           preferred_element_type=jnp.float32)
           + lax.dot_general(fl, ench, (((1,), (0,)), ((), ())),
                             preferred_element_type=jnp.float32))

    @pl.when(i == 0)
    def _():
        cnt_ref[...] = cnt[None, :]
        dwt_ref[...] = dwt

    @pl.when(i > 0)
    def _():
        cnt_ref[...] += cnt[None, :]
        dwt_ref[...] += dwt


def _pass2_body(inp_ref, idx_ref, cnt_ref, dwt_ref, emaw_ref, emacs_ref,
                q_ref, loss_ref, perp_ref, wh_scr, wl_scr, acc_scr):
    i = pl.program_id(0)

    @pl.when(i == 0)
    def _():
        counts = cnt_ref[0, :]
        cs = emacs_ref[0, :] * DECAY_C + (1.0 - DECAY_C) * counts
        n = jnp.sum(cs)
        csn = (cs + EPS_C) / (n + N_EMB * EPS_C) * n
        wt = (emaw_ref[...].T * DECAY_C
              + (1.0 - DECAY_C) * dwt_ref[...]) / csn[None, :]
        wh = wt.astype(jnp.bfloat16)
        wh_scr[...] = wh
        wl_scr[...] = (wt - wh.astype(jnp.float32)).astype(jnp.bfloat16)
        p = counts * (1.0 / T_TOK)
        perp_ref[...] = jnp.exp(-jnp.sum(p * jnp.log(p + 1e-10))).reshape(1, 1)
        acc_scr[...] = jnp.zeros((1, 1), jnp.float32)

    idx = idx_ref[0, 0, :]
    rows = lax.broadcasted_iota(jnp.int32, (N_EMB, BLK), 0)
    encht = (rows == idx[None, :]).astype(jnp.bfloat16)
    qt = (lax.dot_general(wh_scr[...], encht, (((1,), (0,)), ((), ())),
                          preferred_element_type=jnp.float32)
          + lax.dot_general(wl_scr[...], encht, (((1,), (0,)), ((), ())),
                            preferred_element_type=jnp.float32))
    d = qt - inp_ref[...]
    acc_scr[...] = acc_scr[...] + jnp.sum(d * d).reshape(1, 1)
    q_ref[...] = qt

    @pl.when(i == N_BLK - 1)
    def _():
        loss_ref[...] = acc_scr[...] * (COMMIT_C / (T_TOK * DIM))


@jax.jit
def kernel(inputs, embedding_weight, ema_w, ema_cluster_size):
    inp2d = inputs.reshape(DIM, T_TOK)

    dist, enc, idx, cnt, dwt = pl.pallas_call(
        _pass1_body,
        grid=(N_BLK,),
        in_specs=[
            pl.BlockSpec((DIM, BLK), lambda i: (0, i)),
            pl.BlockSpec((N_EMB, DIM), lambda i: (0, 0)),
        ],
        out_specs=[
            pl.BlockSpec((BLK, N_EMB), lambda i: (i, 0)),
            pl.BlockSpec((BLK, N_EMB), lambda i: (i, 0)),
            pl.BlockSpec((1, 1, BLK), lambda i: (i, 0, 0)),
            pl.BlockSpec((1, N_EMB), lambda i: (0, 0)),
            pl.BlockSpec((DIM, N_EMB), lambda i: (0, 0)),
        ],
        out_shape=[
            jax.ShapeDtypeStruct((T_TOK, N_EMB), jnp.float32),
            jax.ShapeDtypeStruct((T_TOK, N_EMB), jnp.float32),
            jax.ShapeDtypeStruct((N_BLK, 1, BLK), jnp.int32),
            jax.ShapeDtypeStruct((1, N_EMB), jnp.float32),
            jax.ShapeDtypeStruct((DIM, N_EMB), jnp.float32),
        ],
        scratch_shapes=[pltpu.VMEM((1, N_EMB), jnp.float32)],
    )(inp2d, embedding_weight)

    q2d, loss, perp = pl.pallas_call(
        _pass2_body,
        grid=(N_BLK,),
        in_specs=[
            pl.BlockSpec((DIM, BLK), lambda i: (0, i)),
            pl.BlockSpec((1, 1, BLK), lambda i: (i, 0, 0)),
            pl.BlockSpec((1, N_EMB), lambda i: (0, 0)),
            pl.BlockSpec((DIM, N_EMB), lambda i: (0, 0)),
            pl.BlockSpec((N_EMB, DIM), lambda i: (0, 0)),
            pl.BlockSpec((1, N_EMB), lambda i: (0, 0)),
        ],
        out_specs=[
            pl.BlockSpec((DIM, BLK), lambda i: (0, i)),
            pl.BlockSpec((1, 1), lambda i: (0, 0)),
            pl.BlockSpec((1, 1), lambda i: (0, 0)),
        ],
        out_shape=[
            jax.ShapeDtypeStruct((DIM, T_TOK), jnp.float32),
            jax.ShapeDtypeStruct((1, 1), jnp.float32),
            jax.ShapeDtypeStruct((1, 1), jnp.float32),
        ],
        scratch_shapes=[
            pltpu.VMEM((DIM, N_EMB), jnp.bfloat16),
            pltpu.VMEM((DIM, N_EMB), jnp.bfloat16),
            pltpu.VMEM((1, 1), jnp.float32),
        ],
    )(inp2d, idx, cnt, dwt, ema_w, ema_cluster_size.reshape(1, N_EMB))

    return (loss[0, 0], q2d.reshape(DIM, 1024, 16), perp[0, 0], enc, dist)


# speedup vs baseline: 1.9648x; 1.1136x over previous
import functools

import jax
import jax.numpy as jnp
from jax import lax
from jax.experimental import pallas as pl
from jax.experimental.pallas import tpu as pltpu

N_EMB = 1024
DIM = 64
T_TOK = 16384
BLK = 1024
N_BLK = T_TOK // BLK
DECAY_C = 0.99
COMMIT_C = 0.25
EPS_C = 1e-05


def _pass1_body(inp_ref, emb_ref, dist_ref, enc_ref, idx_ref, cnt_ref,
                dwt_ref, es_scr):
    i = pl.program_id(0)
    x2d = inp_ref[...]
    e = emb_ref[...]

    @pl.when(i == 0)
    def _():
        es_scr[...] = jnp.sum(e * e, axis=1, keepdims=True).T

    flat = x2d.T
    xs = jnp.sum(flat * flat, axis=1, keepdims=True)
    xe = lax.dot_general(x2d, e, (((0,), (1,)), ((), ())))
    dist = xs + es_scr[...] - 2.0 * xe
    dist_ref[...] = dist
    idx = jnp.argmin(dist, axis=1).astype(jnp.int32)
    idx_ref[...] = idx.reshape(1, 1, BLK)
    cols = lax.broadcasted_iota(jnp.int32, (BLK, N_EMB), 1)
    ench = (cols == idx[:, None]).astype(jnp.bfloat16)
    enc_ref[...] = ench.astype(jnp.float32)
    ones8 = jnp.ones((8, BLK), jnp.bfloat16)
    cnt = lax.dot_general(ones8, ench, (((1,), (0,)), ((), ())),
                          preferred_element_type=jnp.float32)[0]
    fh = x2d.astype(jnp.bfloat16)
    fl = (x2d - fh.astype(jnp.float32)).astype(jnp.bfloat16)
    dwt = (lax.dot_general(fh, ench, (((1,), (0,)), ((), ())),
                           preferred_element_type=jnp.float32)
           + lax.dot_general(fl, ench, (((1,), (0,)), ((), ())),
                             preferred_element_type=jnp.float32))

    @pl.when(i == 0)
    def _():
        cnt_ref[...] = cnt[None, :]
        dwt_ref[...] = dwt

    @pl.when(i > 0)
    def _():
        cnt_ref[...] += cnt[None, :]
        dwt_ref[...] += dwt


def _pass2_body(inp_ref, idx_ref, cnt_ref, dwt_ref, emaw_ref, emacs_ref,
                q_ref, loss_ref, perp_ref, wh_scr, wl_scr, acc_scr):
    i = pl.program_id(0)

    @pl.when(i == 0)
    def _():
        counts = cnt_ref[0, :]
        cs = emacs_ref[0, :] * DECAY_C + (1.0 - DECAY_C) * counts
        n = jnp.sum(cs)
        csn = (cs + EPS_C) / (n + N_EMB * EPS_C) * n
        wt = (emaw_ref[...].T * DECAY_C
              + (1.0 - DECAY_C) * dwt_ref[...]) / csn[None, :]
        wh = wt.astype(jnp.bfloat16)
        wh_scr[...] = wh
        wl_scr[...] = (wt - wh.astype(jnp.float32)).astype(jnp.bfloat16)
        p = counts * (1.0 / T_TOK)
        perp_ref[...] = jnp.exp(-jnp.sum(p * jnp.log(p + 1e-10))).reshape(1, 1)
        acc_scr[...] = jnp.zeros((1, 1), jnp.float32)

    idx = idx_ref[0, 0, :]
    rows = lax.broadcasted_iota(jnp.int32, (N_EMB, BLK), 0)
    encht = (rows == idx[None, :]).astype(jnp.bfloat16)
    qt = (lax.dot_general(wh_scr[...], encht, (((1,), (0,)), ((), ())),
                          preferred_element_type=jnp.float32)
          + lax.dot_general(wl_scr[...], encht, (((1,), (0,)), ((), ())),
                            preferred_element_type=jnp.float32))
    d = qt - inp_ref[...]
    acc_scr[...] = acc_scr[...] + jnp.sum(d * d).reshape(1, 1)
    q_ref[...] = qt

    @pl.when(i == N_BLK - 1)
    def _():
        loss_ref[...] = acc_scr[...] * (COMMIT_C / (T_TOK * DIM))


@jax.jit
def kernel(inputs, embedding_weight, ema_w, ema_cluster_size):
    inp2d = inputs.reshape(DIM, T_TOK)

    dist, enc, idx, cnt, dwt = pl.pallas_call(
        _pass1_body,
        grid=(N_BLK,),
        in_specs=[
            pl.BlockSpec((DIM, BLK), lambda i: (0, i)),
            pl.BlockSpec((N_EMB, DIM), lambda i: (0, 0)),
        ],
        out_specs=[
            pl.BlockSpec((BLK, N_EMB), lambda i: (i, 0)),
            pl.BlockSpec((BLK, N_EMB), lambda i: (i, 0)),
            pl.BlockSpec((1, 1, BLK), lambda i: (i, 0, 0)),
            pl.BlockSpec((1, N_EMB), lambda i: (0, 0)),
            pl.BlockSpec((DIM, N_EMB), lambda i: (0, 0)),
        ],
        out_shape=[
            jax.ShapeDtypeStruct((T_TOK, N_EMB), jnp.float32),
            jax.ShapeDtypeStruct((T_TOK, N_EMB), jnp.float32),
            jax.ShapeDtypeStruct((N_BLK, 1, BLK), jnp.int32),
            jax.ShapeDtypeStruct((1, N_EMB), jnp.float32),
            jax.ShapeDtypeStruct((DIM, N_EMB), jnp.float32),
        ],
        scratch_shapes=[pltpu.VMEM((1, N_EMB), jnp.float32)],
    )(inp2d, embedding_weight)

    q2d, loss, perp = pl.pallas_call(
        _pass2_body,
        grid=(N_BLK,),
        in_specs=[
            pl.BlockSpec((DIM, BLK), lambda i: (0, i)),
            pl.BlockSpec((1, 1, BLK), lambda i: (i, 0, 0)),
            pl.BlockSpec((1, N_EMB), lambda i: (0, 0)),
            pl.BlockSpec((DIM, N_EMB), lambda i: (0, 0)),
            pl.BlockSpec((N_EMB, DIM), lambda i: (0, 0)),
            pl.BlockSpec((1, N_EMB), lambda i: (0, 0)),
        ],
        out_specs=[
            pl.BlockSpec((DIM, BLK), lambda i: (0, i)),
            pl.BlockSpec((1, 1), lambda i: (0, 0)),
            pl.BlockSpec((1, 1), lambda i: (0, 0)),
        ],
        out_shape=[
            jax.ShapeDtypeStruct((DIM, T_TOK), jnp.float32),
            jax.ShapeDtypeStruct((1, 1), jnp.float32),
            jax.ShapeDtypeStruct((1, 1), jnp.float32),
        ],
        scratch_shapes=[
            pltpu.VMEM((DIM, N_EMB), jnp.bfloat16),
            pltpu.VMEM((DIM, N_EMB), jnp.bfloat16),
            pltpu.VMEM((1, 1), jnp.float32),
        ],
    )(inp2d, idx, cnt, dwt, ema_w, ema_cluster_size.reshape(1, N_EMB))

    return (loss[0, 0], q2d.reshape(DIM, 1024, 16), perp[0, 0], enc, dist)


# speedup vs baseline: 1.9851x; 1.0103x over previous
import functools

import jax
import jax.numpy as jnp
from jax import lax
from jax.experimental import pallas as pl
from jax.experimental.pallas import tpu as pltpu

N_EMB = 1024
DIM = 64
T_TOK = 16384
BLK = 2048
N_BLK = T_TOK // BLK
DECAY_C = 0.99
COMMIT_C = 0.25
EPS_C = 1e-05


def _pass1_body(inp_ref, emb_ref, dist_ref, enc_ref, idx_ref, cnt_ref,
                dwt_ref, es_scr):
    i = pl.program_id(0)
    x2d = inp_ref[...]
    e = emb_ref[...]

    @pl.when(i == 0)
    def _():
        es_scr[...] = jnp.sum(e * e, axis=1, keepdims=True).T

    flat = x2d.T
    xs = jnp.sum(flat * flat, axis=1, keepdims=True)
    xe = lax.dot_general(x2d, e, (((0,), (1,)), ((), ())))
    dist = xs + es_scr[...] - 2.0 * xe
    dist_ref[...] = dist
    idx = jnp.argmin(dist, axis=1).astype(jnp.int32)
    idx_ref[...] = idx.reshape(1, 1, BLK)
    cols = lax.broadcasted_iota(jnp.int32, (BLK, N_EMB), 1)
    ench = (cols == idx[:, None]).astype(jnp.bfloat16)
    enc_ref[...] = ench.astype(jnp.float32)
    ones8 = jnp.ones((8, BLK), jnp.bfloat16)
    cnt = lax.dot_general(ones8, ench, (((1,), (0,)), ((), ())),
                          preferred_element_type=jnp.float32)[0]
    fh = x2d.astype(jnp.bfloat16)
    fl = (x2d - fh.astype(jnp.float32)).astype(jnp.bfloat16)
    dwt = (lax.dot_general(fh, ench, (((1,), (0,)), ((), ())),
                           preferred_element_type=jnp.float32)
           + lax.dot_general(fl, ench, (((1,), (0,)), ((), ())),
                             preferred_element_type=jnp.float32))

    @pl.when(i == 0)
    def _():
        cnt_ref[...] = cnt[None, :]
        dwt_ref[...] = dwt

    @pl.when(i > 0)
    def _():
        cnt_ref[...] += cnt[None, :]
        dwt_ref[...] += dwt


def _pass2_body(inp_ref, idx_ref, cnt_ref, dwt_ref, emaw_ref, emacs_ref,
                q_ref, loss_ref, perp_ref, wh_scr, wl_scr, acc_scr):
    i = pl.program_id(0)

    @pl.when(i == 0)
    def _():
        counts = cnt_ref[0, :]
        cs = emacs_ref[0, :] * DECAY_C + (1.0 - DECAY_C) * counts
        n = jnp.sum(cs)
        csn = (cs + EPS_C) / (n + N_EMB * EPS_C) * n
        wt = (emaw_ref[...].T * DECAY_C
              + (1.0 - DECAY_C) * dwt_ref[...]) / csn[None, :]
        wh = wt.astype(jnp.bfloat16)
        wh_scr[...] = wh
        wl_scr[...] = (wt - wh.astype(jnp.float32)).astype(jnp.bfloat16)
        p = counts * (1.0 / T_TOK)
        perp_ref[...] = jnp.exp(-jnp.sum(p * jnp.log(p + 1e-10))).reshape(1, 1)
        acc_scr[...] = jnp.zeros((1, 1), jnp.float32)

    idx = idx_ref[0, 0, :]
    rows = lax.broadcasted_iota(jnp.int32, (N_EMB, BLK), 0)
    encht = (rows == idx[None, :]).astype(jnp.bfloat16)
    qt = (lax.dot_general(wh_scr[...], encht, (((1,), (0,)), ((), ())),
                          preferred_element_type=jnp.float32)
          + lax.dot_general(wl_scr[...], encht, (((1,), (0,)), ((), ())),
                            preferred_element_type=jnp.float32))
    d = qt - inp_ref[...]
    acc_scr[...] = acc_scr[...] + jnp.sum(d * d).reshape(1, 1)
    q_ref[...] = qt

    @pl.when(i == N_BLK - 1)
    def _():
        loss_ref[...] = acc_scr[...] * (COMMIT_C / (T_TOK * DIM))


@jax.jit
def kernel(inputs, embedding_weight, ema_w, ema_cluster_size):
    inp2d = inputs.reshape(DIM, T_TOK)

    dist, enc, idx, cnt, dwt = pl.pallas_call(
        _pass1_body,
        grid=(N_BLK,),
        in_specs=[
            pl.BlockSpec((DIM, BLK), lambda i: (0, i)),
            pl.BlockSpec((N_EMB, DIM), lambda i: (0, 0)),
        ],
        out_specs=[
            pl.BlockSpec((BLK, N_EMB), lambda i: (i, 0)),
            pl.BlockSpec((BLK, N_EMB), lambda i: (i, 0)),
            pl.BlockSpec((1, 1, BLK), lambda i: (i, 0, 0)),
            pl.BlockSpec((1, N_EMB), lambda i: (0, 0)),
            pl.BlockSpec((DIM, N_EMB), lambda i: (0, 0)),
        ],
        out_shape=[
            jax.ShapeDtypeStruct((T_TOK, N_EMB), jnp.float32),
            jax.ShapeDtypeStruct((T_TOK, N_EMB), jnp.float32),
            jax.ShapeDtypeStruct((N_BLK, 1, BLK), jnp.int32),
            jax.ShapeDtypeStruct((1, N_EMB), jnp.float32),
            jax.ShapeDtypeStruct((DIM, N_EMB), jnp.float32),
        ],
        scratch_shapes=[pltpu.VMEM((1, N_EMB), jnp.float32)],
    )(inp2d, embedding_weight)

    q2d, loss, perp = pl.pallas_call(
        _pass2_body,
        grid=(N_BLK,),
        in_specs=[
            pl.BlockSpec((DIM, BLK), lambda i: (0, i)),
            pl.BlockSpec((1, 1, BLK), lambda i: (i, 0, 0)),
            pl.BlockSpec((1, N_EMB), lambda i: (0, 0)),
            pl.BlockSpec((DIM, N_EMB), lambda i: (0, 0)),
            pl.BlockSpec((N_EMB, DIM), lambda i: (0, 0)),
            pl.BlockSpec((1, N_EMB), lambda i: (0, 0)),
        ],
        out_specs=[
            pl.BlockSpec((DIM, BLK), lambda i: (0, i)),
            pl.BlockSpec((1, 1), lambda i: (0, 0)),
            pl.BlockSpec((1, 1), lambda i: (0, 0)),
        ],
        out_shape=[
            jax.ShapeDtypeStruct((DIM, T_TOK), jnp.float32),
            jax.ShapeDtypeStruct((1, 1), jnp.float32),
            jax.ShapeDtypeStruct((1, 1), jnp.float32),
        ],
        scratch_shapes=[
            pltpu.VMEM((DIM, N_EMB), jnp.bfloat16),
            pltpu.VMEM((DIM, N_EMB), jnp.bfloat16),
            pltpu.VMEM((1, 1), jnp.float32),
        ],
    )(inp2d, idx, cnt, dwt, ema_w, ema_cluster_size.reshape(1, N_EMB))

    return (loss[0, 0], q2d.reshape(DIM, 1024, 16), perp[0, 0], enc, dist)
